# baseline (speedup 1.0000x reference)
import functools
import math

import jax
import jax.numpy as jnp
from jax import lax
from jax.experimental import pallas as pl
from jax.experimental.pallas import tpu as pltpu
from jax.experimental.pallas import tpu_sc as plsc

D_MODEL = 128
SCALE = math.sqrt(128.0)
LANES = 16

C = 256
IPR = 128
K = C // IPR
NBUF = 3


def _sc_gather_scale(flat_idx2d, table, *, num_workers, b_per_w):
    n_chunks = b_per_w // C
    rows_pw = b_per_w // IPR
    total = num_workers * b_per_w

    mesh = plsc.VectorSubcoreMesh(core_axis_name="c", subcore_axis_name="s")

    @functools.partial(
        pl.kernel,
        mesh=mesh,
        out_type=jax.ShapeDtypeStruct((total, D_MODEL), jnp.float32),
        scratch_types=[
            pltpu.VMEM((rows_pw, IPR), jnp.int32),
            pltpu.VMEM((C, D_MODEL), jnp.float32),
            pltpu.VMEM((C, D_MODEL), jnp.float32),
            pltpu.VMEM((C, D_MODEL), jnp.float32),
            pltpu.SemaphoreType.DMA,
            pltpu.SemaphoreType.DMA,
            pltpu.SemaphoreType.DMA,
            pltpu.SemaphoreType.DMA,
            pltpu.SemaphoreType.DMA,
            pltpu.SemaphoreType.DMA,
        ],
    )
    def body(idx_hbm, table_hbm, out_hbm, idx_all, rows0, rows1, rows2,
             g0, g1, g2, p0, p1, p2):
        nc = 2
        wid = lax.axis_index("s") * nc + lax.axis_index("c")
        out_row0 = wid * b_per_w

        rows_bufs = (rows0, rows1, rows2)
        g_sems = (g0, g1, g2)
        p_sems = (p0, p1, p2)

        pltpu.sync_copy(idx_hbm.at[pl.ds(wid * rows_pw, rows_pw)], idx_all)

        def gather_descs(i, buf):
            return [
                pltpu.make_async_copy(
                    table_hbm.at[idx_all.at[i * K + k]],
                    rows_bufs[buf].at[pl.ds(k * IPR, IPR)],
                    g_sems[buf],
                )
                for k in range(K)
            ]

        def put_desc(i, buf):
            return pltpu.make_async_copy(
                rows_bufs[buf],
                out_hbm.at[pl.ds(out_row0 + i * C, C)],
                p_sems[buf],
            )

        def scale_buf(buf):
            rows = rows_bufs[buf]

            def srow(r, carry):
                for l in range(D_MODEL // LANES):
                    sl = pl.ds(l * LANES, LANES)
                    rows[r, sl] = rows[r, sl] * SCALE
                return carry

            lax.fori_loop(0, C, srow, 0, unroll=2)

        def step(i, b, fire, wait_prev_put):
            for d in gather_descs(i, b):
                d.wait()
            scale_buf(b)
            put_desc(i, b).start()
            if fire:
                nb = (b + 2) % NBUF
                if wait_prev_put:
                    put_desc(i - 1, nb).wait()
                for d in gather_descs(i + 2, nb):
                    d.start()

        for d in gather_descs(0, 0):
            d.start()
        for d in gather_descs(1, 1):
            d.start()

        step(0, 0, True, False)
        step(1, 1, True, True)
        step(2, 2, True, True)

        def group(j, carry):
            base = 3 * j
            step(base, 0, True, True)
            step(base + 1, 1, True, True)
            step(base + 2, 2, True, True)
            return carry

        lax.fori_loop(1, (n_chunks - 4) // 3, group, 0)

        step(n_chunks - 4, 0, True, True)
        step(n_chunks - 3, 1, True, True)
        step(n_chunks - 2, 2, False, False)
        step(n_chunks - 1, 0, False, False)

        put_desc(n_chunks - 3, 1).wait()
        put_desc(n_chunks - 2, 2).wait()
        put_desc(n_chunks - 1, 0).wait()

    return body(flat_idx2d, table)


def kernel(x, table):
    b, s = x.shape
    total = b * s
    num_workers = 32
    b_per_w = total // num_workers
    flat2d = x.reshape(total // IPR, IPR).astype(jnp.int32)
    out = _sc_gather_scale(flat2d, table,
                           num_workers=num_workers, b_per_w=b_per_w)
    return out.reshape(b, s, D_MODEL)


# speedup vs baseline: 1.0101x; 1.0101x over previous
import functools
import math

import jax
import jax.numpy as jnp
from jax import lax
from jax.experimental import pallas as pl
from jax.experimental.pallas import tpu as pltpu
from jax.experimental.pallas import tpu_sc as plsc

D_MODEL = 128
SCALE = math.sqrt(128.0)
LANES = 16

C = 128
IPR = 128
K = C // IPR
NBUF = 3
NSLOT = 2


def _sc_gather_scale(flat_idx2d, table, *, num_workers, b_per_w):
    n_chunks = b_per_w // C
    rows_pw = b_per_w // IPR
    total = num_workers * b_per_w

    mesh = plsc.VectorSubcoreMesh(core_axis_name="c", subcore_axis_name="s")

    @functools.partial(
        pl.kernel,
        mesh=mesh,
        out_type=jax.ShapeDtypeStruct((total, D_MODEL), jnp.float32),
        scratch_types=[
            pltpu.VMEM((rows_pw, IPR), jnp.int32),
            pltpu.VMEM((C, D_MODEL), jnp.float32),
            pltpu.VMEM((C, D_MODEL), jnp.float32),
            pltpu.VMEM((C, D_MODEL), jnp.float32),
            pltpu.VMEM_SHARED((16, NSLOT, C, D_MODEL), jnp.float32),
            pltpu.SemaphoreType.DMA,
            pltpu.SemaphoreType.DMA,
            pltpu.SemaphoreType.DMA,
            pltpu.SemaphoreType.DMA,
            pltpu.SemaphoreType.DMA,
        ],
    )
    def body(idx_hbm, table_hbm, out_hbm, idx_all, rows0, rows1, rows2,
             stage, g0, g1, g2, p0, p1):
        nc = 2
        wid = lax.axis_index("s") * nc + lax.axis_index("c")
        sid = lax.axis_index("s")
        out_row0 = wid * b_per_w

        rows_bufs = (rows0, rows1, rows2)
        g_sems = (g0, g1, g2)
        p_sems = (p0, p1)

        pltpu.sync_copy(idx_hbm.at[pl.ds(wid * rows_pw, rows_pw)], idx_all)

        def gather_descs(i, b):
            return [
                pltpu.make_async_copy(
                    table_hbm.at[idx_all.at[i * K + k]],
                    rows_bufs[b].at[pl.ds(k * IPR, IPR)],
                    g_sems[b],
                )
                for k in range(K)
            ]

        def put_desc(i, t):
            return pltpu.make_async_copy(
                stage.at[sid, t],
                out_hbm.at[pl.ds(out_row0 + i * C, C)],
                p_sems[t],
            )

        def scale_buf(b):
            rows = rows_bufs[b]

            def srow(r, carry):
                for l in range(D_MODEL // LANES):
                    sl = pl.ds(l * LANES, LANES)
                    rows[r, sl] = rows[r, sl] * SCALE
                return carry

            lax.fori_loop(0, C, srow, 0, unroll=2)

        def step(i, b, t, fire=True, wait_put=True, start_prev_put=True):
            for d in gather_descs(i, b):
                d.wait()
            scale_buf(b)
            if wait_put:
                put_desc(i - NSLOT, t).wait()
            pltpu.sync_copy(rows_bufs[b], stage.at[sid, t])
            if start_prev_put:
                put_desc(i - 1, 1 - t).start()
            if fire:
                for d in gather_descs(i + NBUF, b):
                    d.start()

        for i in range(NBUF):
            for d in gather_descs(i, i):
                d.start()

        step(0, 0, 0, wait_put=False, start_prev_put=False)
        step(1, 1, 1, wait_put=False)
        for i in range(2, 6):
            step(i, i % NBUF, i % NSLOT)

        def group(j, carry):
            base = 6 * j
            for u in range(6):
                step(base + u, u % NBUF, u % NSLOT)
            return carry

        n_steady_groups = (n_chunks - NBUF - 6) // 6
        lax.fori_loop(1, 1 + n_steady_groups, group, 0)

        for i in range(6 * (1 + n_steady_groups), n_chunks):
            step(i, i % NBUF, i % NSLOT, fire=(i < n_chunks - NBUF))

        put_desc(n_chunks - 1, (n_chunks - 1) % NSLOT).start()
        put_desc(n_chunks - 2, (n_chunks - 2) % NSLOT).wait()
        put_desc(n_chunks - 1, (n_chunks - 1) % NSLOT).wait()

    return body(flat_idx2d, table)


def kernel(x, table):
    b, s = x.shape
    total = b * s
    num_workers = 32
    b_per_w = total // num_workers
    flat2d = x.reshape(total // IPR, IPR).astype(jnp.int32)
    out = _sc_gather_scale(flat2d, table,
                           num_workers=num_workers, b_per_w=b_per_w)
    return out.reshape(b, s, D_MODEL)
